# baseline (speedup 1.0000x reference)
import jax
import jax.numpy as jnp
from jax import lax
from jax.experimental import pallas as pl
from jax.experimental.pallas import tpu as pltpu
from jax.experimental.pallas import tpu_sc as plsc

N_CLS = 1000
N_CTX = 16
CTX_DIM = 512
SEQ = 77
SUF = SEQ - 1 - N_CTX
B = 1024
TOK_PAD = 96
NC = 2
NS = 16
NW = NC * NS
RPW = B // NW
CC = 2
CTX_CHUNKS = RPW // CC


def _gather_table(tbl, idx_rows, nchunks, rows_per_chunk, buf, dst_fn, sem_g, sem_o):
    outh = [None, None]
    for j in range(nchunks):
        b = j % 2
        if outh[b] is not None:
            outh[b].wait()
        pltpu.async_copy(tbl.at[idx_rows.at[j]], buf.at[b], sem_g).wait()
        outh[b] = pltpu.async_copy(buf.at[b], dst_fn(j), sem_o)
    for h in outh:
        if h is not None:
            h.wait()


def _body(pre_n, ctx_n, suf_n, pre_p, ctx_p, suf_p, tok_n, tok_p,
          cls1, cls_c, cls_s,
          out, tok_out,
          idx_all, idx_c, idx_s, tok_buf, pre_buf, ctx_buf, suf_buf,
          sem_g, sem_o):
    wid = lax.axis_index("s") * NC + lax.axis_index("c")
    base = wid * RPW
    pltpu.sync_copy(cls1.at[pl.ds(base, RPW)], idx_all)
    pltpu.sync_copy(cls_c.at[pl.ds(wid * CTX_CHUNKS, CTX_CHUNKS), :], idx_c)
    pltpu.sync_copy(cls_s.at[pl.ds(base, RPW), :], idx_s)

    for p, (pre, ctx, suf, tok) in enumerate(
            ((pre_n, ctx_n, suf_n, tok_n), (pre_p, ctx_p, suf_p, tok_p))):
        pltpu.async_copy(tok.at[idx_all], tok_buf, sem_g).wait()
        pltpu.sync_copy(tok_buf, tok_out.at[p, pl.ds(base, RPW), :])
        pltpu.async_copy(pre.at[idx_all], pre_buf, sem_g).wait()
        pltpu.sync_copy(pre_buf, out.at[p, pl.ds(base, RPW), pl.ds(0, 1), :])
        _gather_table(
            ctx, idx_c, CTX_CHUNKS, CC, ctx_buf,
            lambda j: out.at[p, pl.ds(base + j * CC, CC), pl.ds(1, N_CTX), :],
            sem_g, sem_o)
        _gather_table(
            suf, idx_s, RPW, 1, suf_buf,
            lambda j: out.at[p, pl.ds(base + j, 1), pl.ds(1 + N_CTX, SUF), :],
            sem_g, sem_o)


def kernel(ctx_pos, ctx_neg, token_prefix_pos, token_suffix_pos,
           token_prefix_neg, token_suffix_neg, tokenized_prompts, cls_id):
    tok_padded = jnp.pad(tokenized_prompts, ((0, 0), (0, TOK_PAD - SEQ)))
    tok_neg = tok_padded[:N_CLS]
    tok_pos = tok_padded[N_CLS:]
    cls1 = cls_id
    cls_c = cls_id.reshape(B // CC, CC)
    cls_s = cls_id.reshape(B, 1)

    k = pl.kernel(
        _body,
        out_type=(
            jax.ShapeDtypeStruct((2, B, SEQ, CTX_DIM), jnp.float32),
            jax.ShapeDtypeStruct((2, B, TOK_PAD), jnp.int32),
        ),
        mesh=plsc.VectorSubcoreMesh(core_axis_name="c", subcore_axis_name="s",
                                    num_cores=NC, num_subcores=NS),
        compiler_params=pltpu.CompilerParams(use_tc_tiling_on_sc=False),
        scratch_types=[
            pltpu.VMEM((RPW,), jnp.int32),
            pltpu.VMEM((CTX_CHUNKS, CC), jnp.int32),
            pltpu.VMEM((RPW, 1), jnp.int32),
            pltpu.VMEM((RPW, TOK_PAD), jnp.int32),
            pltpu.VMEM((RPW, 1, CTX_DIM), jnp.float32),
            pltpu.VMEM((2, CC, N_CTX, CTX_DIM), jnp.float32),
            pltpu.VMEM((2, 1, SUF, CTX_DIM), jnp.float32),
            pltpu.SemaphoreType.DMA,
            pltpu.SemaphoreType.DMA,
        ],
    )
    prompts4, tok3 = k(token_prefix_neg, ctx_neg, token_suffix_neg,
                       token_prefix_pos, ctx_pos, token_suffix_pos,
                       tok_neg, tok_pos, cls1, cls_c, cls_s)
    return (prompts4.reshape(2 * B, SEQ, CTX_DIM),
            tok3.reshape(2 * B, TOK_PAD)[:, :SEQ])


# speedup vs baseline: 1.0086x; 1.0086x over previous
import jax
import jax.numpy as jnp
from jax import lax
from jax.experimental import pallas as pl
from jax.experimental.pallas import tpu as pltpu
from jax.experimental.pallas import tpu_sc as plsc

N_CLS = 1000
N_CTX = 16
CTX_DIM = 512
SEQ = 77
SUF = SEQ - 1 - N_CTX
B = 1024
TOK_PAD = 96
NC = 2
NS = 16
NW = NC * NS
RPW = B // NW
CC = 2
CTX_CHUNKS = RPW // CC


def _gather_table(tbl, idx_rows, nchunks, buf, dst_fn, sems_g, sems_o):
    gh = [None, None]
    outh = [None, None]
    gh[0] = pltpu.async_copy(tbl.at[idx_rows.at[0]], buf.at[0], sems_g[0])
    for j in range(nchunks):
        b = j % 2
        nb = (j + 1) % 2
        if j + 1 < nchunks:
            if outh[nb] is not None:
                outh[nb].wait()
            gh[nb] = pltpu.async_copy(tbl.at[idx_rows.at[j + 1]], buf.at[nb],
                                      sems_g[nb])
        gh[b].wait()
        outh[b] = pltpu.async_copy(buf.at[b], dst_fn(j), sems_o[b])
    for h in outh:
        if h is not None:
            h.wait()


def _body(pre_n, ctx_n, suf_n, pre_p, ctx_p, suf_p, tok_n, tok_p,
          cls1, cls_c, cls_s,
          out, tok_out,
          idx_all, idx_c, idx_s, tok_buf, pre_buf, ctx_buf, suf_buf,
          sem_g0, sem_g1, sem_o0, sem_o1, sem_t, sem_p):
    wid = lax.axis_index("s") * NC + lax.axis_index("c")
    base = wid * RPW
    pltpu.sync_copy(cls1.at[pl.ds(base, RPW)], idx_all)
    pltpu.sync_copy(cls_c.at[pl.ds(wid * CTX_CHUNKS, CTX_CHUNKS), :], idx_c)
    pltpu.sync_copy(cls_s.at[pl.ds(base, RPW), :], idx_s)

    for p, (pre, ctx, suf, tok) in enumerate(
            ((pre_n, ctx_n, suf_n, tok_n), (pre_p, ctx_p, suf_p, tok_p))):
        gt = pltpu.async_copy(tok.at[idx_all], tok_buf, sem_t)
        gp = pltpu.async_copy(pre.at[idx_all], pre_buf, sem_p)
        gt.wait()
        ot = pltpu.async_copy(tok_buf, tok_out.at[p, pl.ds(base, RPW), :],
                              sem_t)
        gp.wait()
        op = pltpu.async_copy(
            pre_buf, out.at[p, pl.ds(base, RPW), pl.ds(0, 1), :], sem_p)
        _gather_table(
            ctx, idx_c, CTX_CHUNKS, ctx_buf,
            lambda j: out.at[p, pl.ds(base + j * CC, CC), pl.ds(1, N_CTX), :],
            (sem_g0, sem_g1), (sem_o0, sem_o1))
        _gather_table(
            suf, idx_s, RPW, suf_buf,
            lambda j: out.at[p, pl.ds(base + j, 1), pl.ds(1 + N_CTX, SUF), :],
            (sem_g0, sem_g1), (sem_o0, sem_o1))
        ot.wait()
        op.wait()


def kernel(ctx_pos, ctx_neg, token_prefix_pos, token_suffix_pos,
           token_prefix_neg, token_suffix_neg, tokenized_prompts, cls_id):
    tok_padded = jnp.pad(tokenized_prompts, ((0, 0), (0, TOK_PAD - SEQ)))
    tok_neg = tok_padded[:N_CLS]
    tok_pos = tok_padded[N_CLS:]
    cls1 = cls_id
    cls_c = cls_id.reshape(B // CC, CC)
    cls_s = cls_id.reshape(B, 1)

    k = pl.kernel(
        _body,
        out_type=(
            jax.ShapeDtypeStruct((2, B, SEQ, CTX_DIM), jnp.float32),
            jax.ShapeDtypeStruct((2, B, TOK_PAD), jnp.int32),
        ),
        mesh=plsc.VectorSubcoreMesh(core_axis_name="c", subcore_axis_name="s",
                                    num_cores=NC, num_subcores=NS),
        compiler_params=pltpu.CompilerParams(use_tc_tiling_on_sc=False),
        scratch_types=[
            pltpu.VMEM((RPW,), jnp.int32),
            pltpu.VMEM((CTX_CHUNKS, CC), jnp.int32),
            pltpu.VMEM((RPW, 1), jnp.int32),
            pltpu.VMEM((RPW, TOK_PAD), jnp.int32),
            pltpu.VMEM((RPW, 1, CTX_DIM), jnp.float32),
            pltpu.VMEM((2, CC, N_CTX, CTX_DIM), jnp.float32),
            pltpu.VMEM((2, 1, SUF, CTX_DIM), jnp.float32),
            pltpu.SemaphoreType.DMA,
            pltpu.SemaphoreType.DMA,
            pltpu.SemaphoreType.DMA,
            pltpu.SemaphoreType.DMA,
            pltpu.SemaphoreType.DMA,
            pltpu.SemaphoreType.DMA,
        ],
    )
    prompts4, tok3 = k(token_prefix_neg, ctx_neg, token_suffix_neg,
                       token_prefix_pos, ctx_pos, token_suffix_pos,
                       tok_neg, tok_pos, cls1, cls_c, cls_s)
    return (prompts4.reshape(2 * B, SEQ, CTX_DIM),
            tok3.reshape(2 * B, TOK_PAD)[:, :SEQ])


# speedup vs baseline: 6.3762x; 6.3216x over previous
import jax
import jax.numpy as jnp
from jax import lax
from jax.experimental import pallas as pl
from jax.experimental.pallas import tpu as pltpu
from jax.experimental.pallas import tpu_sc as plsc

N_CLS = 1000
N_CTX = 16
CTX_DIM = 512
SEQ = 77
SUF = SEQ - 1 - N_CTX
B = 1024
TOK_PAD = 128
NC = 2
NS = 16
NW = NC * NS
RPW = B // NW
SLOTS = 4


def _run_jobs(jobs, bufs, gsems, osems):
    n = len(jobs)
    gh = [None] * SLOTS
    outh = [None] * SLOTS

    def issue(j):
        b = j % SLOTS
        if outh[b] is not None:
            outh[b].wait()
            outh[b] = None
        src, idxr, _ = jobs[j]
        gh[b] = pltpu.async_copy(src.at[idxr], bufs.at[b], gsems[b])

    for j in range(min(SLOTS, n)):
        issue(j)
    for j in range(n):
        b = j % SLOTS
        gh[b].wait()
        outh[b] = pltpu.async_copy(bufs.at[b], jobs[j][2], osems[b])
        if j + SLOTS < n:
            issue(j + SLOTS)
    for h in outh:
        if h is not None:
            h.wait()


def _body(pre_n, ctx_n, suf_n, pre_p, ctx_p, suf_p, tok_n, tok_p, cls1,
          out, tok_out,
          idx_all, idx_c, idx_s, tok_buf, bufs,
          gs0, gs1, gs2, gs3, os0, os1, os2, os3, sem_t):
    wid = lax.axis_index("s") * NC + lax.axis_index("c")
    base = pl.multiple_of(wid * RPW, RPW)
    pltpu.sync_copy(cls1.at[pl.ds(base, RPW)], idx_all)

    for h in range(RPW // 16):
        c = idx_all[pl.ds(16 * h, 16)]
        for s in range(N_CTX):
            idx_c[s, pl.ds(16 * h, 16)] = c * N_CTX + s
        for s in range(SUF):
            idx_s[s, pl.ds(16 * h, 16)] = c + N_CLS * s

    gsems = (gs0, gs1, gs2, gs3)
    osems = (os0, os1, os2, os3)
    for p, (pre, ctx, suf, tok) in enumerate(
            ((pre_n, ctx_n, suf_n, tok_n), (pre_p, ctx_p, suf_p, tok_p))):
        gt = pltpu.async_copy(tok.at[idx_all], tok_buf, sem_t)
        jobs = [(pre, idx_all, out.at[0, p, pl.ds(base, RPW), :])]
        for s in range(N_CTX):
            jobs.append((ctx, idx_c.at[s],
                         out.at[1 + s, p, pl.ds(base, RPW), :]))
        for s in range(SUF):
            jobs.append((suf, idx_s.at[s],
                         out.at[1 + N_CTX + s, p, pl.ds(base, RPW), :]))
        _run_jobs(jobs, bufs, gsems, osems)
        gt.wait()
        pltpu.async_copy(tok_buf, tok_out.at[p, pl.ds(base, RPW), :],
                         sem_t).wait()


def kernel(ctx_pos, ctx_neg, token_prefix_pos, token_suffix_pos,
           token_prefix_neg, token_suffix_neg, tokenized_prompts, cls_id):
    pre_n2 = token_prefix_neg.reshape(N_CLS, CTX_DIM)
    pre_p2 = token_prefix_pos.reshape(N_CLS, CTX_DIM)
    ctx_n2 = ctx_neg.reshape(N_CLS * N_CTX, CTX_DIM)
    ctx_p2 = ctx_pos.reshape(N_CLS * N_CTX, CTX_DIM)
    suf_n2 = jnp.transpose(token_suffix_neg, (1, 0, 2)).reshape(
        SUF * N_CLS, CTX_DIM)
    suf_p2 = jnp.transpose(token_suffix_pos, (1, 0, 2)).reshape(
        SUF * N_CLS, CTX_DIM)
    tok_padded = jnp.pad(tokenized_prompts, ((0, 0), (0, TOK_PAD - SEQ)))
    tok_neg = tok_padded[:N_CLS]
    tok_pos = tok_padded[N_CLS:]

    k = pl.kernel(
        _body,
        out_type=(
            jax.ShapeDtypeStruct((SEQ, 2, B, CTX_DIM), jnp.float32),
            jax.ShapeDtypeStruct((2, B, TOK_PAD), jnp.int32),
        ),
        mesh=plsc.VectorSubcoreMesh(core_axis_name="c", subcore_axis_name="s",
                                    num_cores=NC, num_subcores=NS),
        scratch_types=[
            pltpu.VMEM((RPW,), jnp.int32),
            pltpu.VMEM((N_CTX, RPW), jnp.int32),
            pltpu.VMEM((SUF, RPW), jnp.int32),
            pltpu.VMEM((RPW, TOK_PAD), jnp.int32),
            pltpu.VMEM((SLOTS, RPW, CTX_DIM), jnp.float32),
            pltpu.SemaphoreType.DMA, pltpu.SemaphoreType.DMA,
            pltpu.SemaphoreType.DMA, pltpu.SemaphoreType.DMA,
            pltpu.SemaphoreType.DMA, pltpu.SemaphoreType.DMA,
            pltpu.SemaphoreType.DMA, pltpu.SemaphoreType.DMA,
            pltpu.SemaphoreType.DMA,
        ],
    )
    prompts4, tok3 = k(pre_n2, ctx_n2, suf_n2, pre_p2, ctx_p2, suf_p2,
                       tok_neg, tok_pos, cls_id)
    prompts = jnp.transpose(prompts4, (1, 2, 0, 3)).reshape(
        2 * B, SEQ, CTX_DIM)
    return prompts, tok3.reshape(2 * B, TOK_PAD)[:, :SEQ]
